# baseline (speedup 1.0000x reference)
import functools

import jax
import jax.numpy as jnp
from jax import lax
from jax.experimental import pallas as pl
from jax.experimental.pallas import tpu as pltpu
from jax.experimental.pallas import tpu_sc as plsc

N = 10000
NP = 10240
E = 160000
D = 256
DH = 128
B = 128
NCHUNK = E // B
NTILES = 16
ROWS_PER_TILE = NP // NTILES
ZROWS = B

NCH_BASE = NCHUNK // NTILES
NCH_REM = NCHUNK % NTILES
NCH_CEIL = NCH_BASE + (2 if NCH_REM else 0)
NCH_CEIL6 = ((NCH_BASE + 1) + 5) // 6 * 6

_mesh = plsc.VectorSubcoreMesh(core_axis_name="c", subcore_axis_name="s")


def _sc_agg_body(x3, ei_h, agg3, ws_out,
                 idx3_0, idx3_1, idx3_2, w_a, w_b, rows_a, rows_b, zws_v,
                 acc_sh, ws_sh, sem_a, sem_b, isem_0, isem_1, isem_2,
                 ssem_a, ssem_b):
    c = lax.axis_index("c")
    s = lax.axis_index("s")

    def zrow(i, carry):
        for k in range(DH // 16):
            rows_a[i, k * 16:(k + 1) * 16] = jnp.zeros((16,), jnp.float32)
        return carry
    lax.fori_loop(0, ZROWS, zrow, 0)

    def zws_row(i, carry):
        zws_v[pl.ds(i * 16, 16)] = jnp.zeros((16,), jnp.float32)
        return carry
    lax.fori_loop(0, ROWS_PER_TILE // 16, zws_row, 0)

    base_rows = s * ROWS_PER_TILE
    for kk in range(ROWS_PER_TILE // ZROWS):
        pltpu.sync_copy(rows_a, acc_sh.at[pl.ds(base_rows + kk * ZROWS, ZROWS)])
    pltpu.sync_copy(zws_v, ws_sh.at[pl.ds(base_rows, ROWS_PER_TILE)])

    plsc.subcore_barrier()

    nch = NCH_BASE + jnp.where(s < NCH_REM, 1, 0)
    rbufs = ((w_a, rows_a, sem_a), (w_b, rows_b, sem_b))
    ibufs = ((idx3_0, isem_0), (idx3_1, isem_1), (idx3_2, isem_2))

    def ei_slice(jj):
        return ei_h.at[:, pl.ds((s + jj * NTILES) * B, B)]

    def issue_idx(jj, ibuf):
        idx3_v, isem = ibuf
        pltpu.async_copy(ei_slice(jj), idx3_v, isem)

    def fire_gather(jj, ibuf, rbuf):
        idx3_v, isem = ibuf
        _, rows_v, sem = rbuf
        pltpu.make_async_copy(ei_slice(jj), idx3_v, isem).wait()
        pltpu.async_copy(x3.at[c].at[idx3_v.at[0]], rows_v, sem)

    issue_idx(0, ibufs[0])
    issue_idx(1, ibufs[1])
    fire_gather(0, ibufs[0], rbufs[0])

    @pl.loop(0, NCH_CEIL6, step=6)
    def _chunks(j):
        for u in range(6):
            w_v, rows_v, sem = rbufs[u % 2]
            ssem = (ssem_a, ssem_b)[u % 2]
            w_o, rows_o, _ = rbufs[(u + 1) % 2]
            ssem_o = (ssem_a, ssem_b)[(u + 1) % 2]
            idx3_v, isem = ibufs[u % 3]
            i1buf = ibufs[(u + 1) % 3]
            i2buf = ibufs[(u + 2) % 3]
            jj = j + u

            @pl.when(jj < nch)
            def _():
                pltpu.make_async_copy(
                    x3.at[c].at[idx3_v.at[0]], rows_v, sem).wait()

                @pl.when(jj >= 1)
                def _():
                    pltpu.make_async_copy(
                        rows_o, acc_sh.at[i1buf[0].at[1]], ssem_o).wait()

                @pl.when(jj + 2 < nch)
                def _():
                    issue_idx(jj + 2, i2buf)

                @pl.when(jj + 1 < nch)
                def _():
                    fire_gather(jj + 1, i1buf, rbufs[(u + 1) % 2])

                def grp(g, rcarry):
                    wvec = lax.bitcast_convert_type(
                        idx3_v[2, pl.ds(g * 16, 16)], jnp.float32)
                    w_v[pl.ds(g * 16, 16)] = wvec
                    for r in range(16):
                        wr = wvec[r]
                        row = g * 16 + r
                        for k in range(DH // 16):
                            sl = pl.ds(k * 16, 16)
                            rows_v[row, sl] = rows_v[row, sl] * wr
                    return rcarry
                lax.fori_loop(0, B // 16, grp, 0)

                pltpu.async_copy(rows_v, acc_sh.at[idx3_v.at[1]], ssem,
                                 add=True)

                @pl.when((jj & 1) == c)
                def _():
                    pltpu.sync_copy(w_v, ws_sh.at[idx3_v.at[1]], add=True)

    @pl.when(((nch - 1) & 1) == 0)
    def _():
        pltpu.make_async_copy(
            rows_a, acc_sh.at[idx3_0.at[1]], ssem_a).wait()

    @pl.when(((nch - 1) & 1) == 1)
    def _():
        pltpu.make_async_copy(
            rows_b, acc_sh.at[idx3_0.at[1]], ssem_b).wait()

    plsc.subcore_barrier()

    pltpu.sync_copy(acc_sh.at[pl.ds(base_rows, ROWS_PER_TILE)],
                    agg3.at[c].at[pl.ds(base_rows, ROWS_PER_TILE)])
    pltpu.sync_copy(ws_sh.at[pl.ds(base_rows, ROWS_PER_TILE)],
                    ws_out.at[c].at[pl.ds(base_rows, ROWS_PER_TILE)])


_sc_agg = functools.partial(
    pl.kernel,
    out_type=(jax.ShapeDtypeStruct((2, NP, DH), jnp.float32),
              jax.ShapeDtypeStruct((2, NP), jnp.float32)),
    mesh=_mesh,
    scratch_types=[
        pltpu.VMEM((3, B), jnp.int32),
        pltpu.VMEM((3, B), jnp.int32),
        pltpu.VMEM((3, B), jnp.int32),
        pltpu.VMEM((B,), jnp.float32),
        pltpu.VMEM((B,), jnp.float32),
        pltpu.VMEM((B, DH), jnp.float32),
        pltpu.VMEM((B, DH), jnp.float32),
        pltpu.VMEM((ROWS_PER_TILE,), jnp.float32),
        pltpu.VMEM_SHARED((NP, DH), jnp.float32),
        pltpu.VMEM_SHARED((NP,), jnp.float32),
        pltpu.SemaphoreType.DMA,
        pltpu.SemaphoreType.DMA,
        pltpu.SemaphoreType.DMA,
        pltpu.SemaphoreType.DMA,
        pltpu.SemaphoreType.DMA,
        pltpu.SemaphoreType.DMA,
        pltpu.SemaphoreType.DMA,
    ],
)(_sc_agg_body)


def _dense_body(h_ref, agg_ref, ws_ref, W_ref, b_ref, out_ref):
    hl = h_ref[0]
    hh = h_ref[1]
    inv = 1.0 / (ws_ref[0] + ws_ref[1] + 1e-9)
    al = agg_ref[0] * inv
    ah = agg_ref[1] * inv
    W = W_ref[...]
    z = (jnp.dot(hl, W[0:128, :], preferred_element_type=jnp.float32)
         + jnp.dot(hh, W[128:256, :], preferred_element_type=jnp.float32)
         + jnp.dot(al, W[256:384, :], preferred_element_type=jnp.float32)
         + jnp.dot(ah, W[384:512, :], preferred_element_type=jnp.float32)
         + b_ref[...])
    z = jnp.maximum(z, 0.0)
    z = z / (jnp.sqrt(jnp.sum(z * z, axis=1, keepdims=True)) + 1e-9)
    out_ref[0, :, :] = z[:, :DH]
    out_ref[1, :, :] = z[:, DH:]


def _dense_last_body(h_ref, agg_ref, ws_ref, W_ref, b_ref, out_ref):
    hl = h_ref[0]
    hh = h_ref[1]
    inv = 1.0 / (ws_ref[0] + ws_ref[1] + 1e-9)
    al = agg_ref[0] * inv
    ah = agg_ref[1] * inv
    W = W_ref[...]
    z = (jnp.dot(hl, W[0:128, :], preferred_element_type=jnp.float32)
         + jnp.dot(hh, W[128:256, :], preferred_element_type=jnp.float32)
         + jnp.dot(al, W[256:384, :], preferred_element_type=jnp.float32)
         + jnp.dot(ah, W[384:512, :], preferred_element_type=jnp.float32)
         + b_ref[...])
    z = jnp.maximum(z, 0.0)
    z = z / (jnp.sqrt(jnp.sum(z * z, axis=1, keepdims=True)) + 1e-9)
    out_ref[...] = z


_R = 256

_dense = pl.pallas_call(
    _dense_body,
    grid=(NP // _R,),
    in_specs=[
        pl.BlockSpec((2, _R, DH), lambda i: (0, i, 0)),
        pl.BlockSpec((2, _R, DH), lambda i: (0, i, 0)),
        pl.BlockSpec((2, _R, 1), lambda i: (0, i, 0)),
        pl.BlockSpec((2 * D, D), lambda i: (0, 0)),
        pl.BlockSpec((1, D), lambda i: (0, 0)),
    ],
    out_specs=pl.BlockSpec((2, _R, DH), lambda i: (0, i, 0)),
    out_shape=jax.ShapeDtypeStruct((2, NP, DH), jnp.float32),
)


_dense_last = pl.pallas_call(
    _dense_last_body,
    grid=(NP // _R,),
    in_specs=[
        pl.BlockSpec((2, _R, DH), lambda i: (0, i, 0)),
        pl.BlockSpec((2, _R, DH), lambda i: (0, i, 0)),
        pl.BlockSpec((2, _R, 1), lambda i: (0, i, 0)),
        pl.BlockSpec((2 * D, D), lambda i: (0, 0)),
        pl.BlockSpec((1, D), lambda i: (0, 0)),
    ],
    out_specs=pl.BlockSpec((_R, D), lambda i: (i, 0)),
    out_shape=jax.ShapeDtypeStruct((NP, D), jnp.float32),
)


def kernel(x, edge_index, edge_weight, W0, b0, W1, b1):
    h3 = jnp.pad(x, ((0, NP - N), (0, 0))).reshape(NP, 2, DH).transpose(1, 0, 2)
    ei3 = jnp.concatenate(
        [edge_index,
         lax.bitcast_convert_type(edge_weight, jnp.int32)[None]], axis=0)

    agg3, ws = _sc_agg(h3, ei3)
    h3 = _dense(h3, agg3, ws.reshape(2, NP, 1), W0, b0.reshape(1, D))
    agg3, ws = _sc_agg(h3, ei3)
    out = _dense_last(h3, agg3, ws.reshape(2, NP, 1), W1, b1.reshape(1, D))
    return out[:N]


# speedup vs baseline: 1.0027x; 1.0027x over previous
import functools

import jax
import jax.numpy as jnp
from jax import lax
from jax.experimental import pallas as pl
from jax.experimental.pallas import tpu as pltpu
from jax.experimental.pallas import tpu_sc as plsc

N = 10000
NP = 10240
E = 160000
D = 256
DH = 128
B = 128
NCHUNK = E // B
NTILES = 16
ROWS_PER_TILE = NP // NTILES
ZROWS = B

NCH_BASE = NCHUNK // NTILES
NCH_REM = NCHUNK % NTILES
NCH_CEIL = NCH_BASE + (2 if NCH_REM else 0)
NCH_CEIL6 = ((NCH_BASE + 1) + 5) // 6 * 6

_mesh = plsc.VectorSubcoreMesh(core_axis_name="c", subcore_axis_name="s")


def _sc_agg_body(x3, ei_h, agg3, ws_out,
                 idx3_0, idx3_1, idx3_2, w_a, w_b, rows_a, rows_b, zws_v,
                 acc_sh, ws_sh, sem_a, sem_b, isem_0, isem_1, isem_2):
    c = lax.axis_index("c")
    s = lax.axis_index("s")

    def zrow(i, carry):
        for k in range(DH // 16):
            rows_a[i, k * 16:(k + 1) * 16] = jnp.zeros((16,), jnp.float32)
        return carry
    lax.fori_loop(0, ZROWS, zrow, 0)

    def zws_row(i, carry):
        zws_v[pl.ds(i * 16, 16)] = jnp.zeros((16,), jnp.float32)
        return carry
    lax.fori_loop(0, ROWS_PER_TILE // 16, zws_row, 0)

    base_rows = s * ROWS_PER_TILE
    for kk in range(ROWS_PER_TILE // ZROWS):
        pltpu.sync_copy(rows_a, acc_sh.at[pl.ds(base_rows + kk * ZROWS, ZROWS)])
    pltpu.sync_copy(zws_v, ws_sh.at[pl.ds(base_rows, ROWS_PER_TILE)])

    plsc.subcore_barrier()

    nch = NCH_BASE + jnp.where(s < NCH_REM, 1, 0)
    rbufs = ((w_a, rows_a, sem_a), (w_b, rows_b, sem_b))
    ibufs = ((idx3_0, isem_0), (idx3_1, isem_1), (idx3_2, isem_2))

    def ei_slice(jj):
        return ei_h.at[:, pl.ds((s + jj * NTILES) * B, B)]

    def issue_idx(jj, ibuf):
        idx3_v, isem = ibuf
        pltpu.async_copy(ei_slice(jj), idx3_v, isem)

    def fire_gather(jj, ibuf, rbuf):
        idx3_v, isem = ibuf
        _, rows_v, sem = rbuf
        pltpu.make_async_copy(ei_slice(jj), idx3_v, isem).wait()
        pltpu.async_copy(x3.at[c].at[idx3_v.at[0]], rows_v, sem)

    issue_idx(0, ibufs[0])
    issue_idx(1, ibufs[1])
    fire_gather(0, ibufs[0], rbufs[0])
    fire_gather(1, ibufs[1], rbufs[1])

    @pl.loop(0, NCH_CEIL6, step=6)
    def _chunks(j):
        for u in range(6):
            w_v, rows_v, sem = rbufs[u % 2]
            idx3_v, isem = ibufs[u % 3]
            i2buf = ibufs[(u + 2) % 3]
            jj = j + u

            @pl.when(jj < nch)
            def _():
                @pl.when(jj + 2 < nch)
                def _():
                    issue_idx(jj + 2, i2buf)

                pltpu.make_async_copy(
                    x3.at[c].at[idx3_v.at[0]], rows_v, sem).wait()

                def grp(g, rcarry):
                    wvec = lax.bitcast_convert_type(
                        idx3_v[2, pl.ds(g * 16, 16)], jnp.float32)
                    w_v[pl.ds(g * 16, 16)] = wvec
                    for r in range(16):
                        wr = wvec[r]
                        row = g * 16 + r
                        for k in range(DH // 16):
                            sl = pl.ds(k * 16, 16)
                            rows_v[row, sl] = rows_v[row, sl] * wr
                    return rcarry
                lax.fori_loop(0, B // 16, grp, 0)

                pltpu.sync_copy(rows_v, acc_sh.at[idx3_v.at[1]], add=True)

                @pl.when((jj & 1) == c)
                def _():
                    pltpu.sync_copy(w_v, ws_sh.at[idx3_v.at[1]], add=True)

                @pl.when(jj + 2 < nch)
                def _():
                    fire_gather(jj + 2, i2buf, rbufs[u % 2])

    plsc.subcore_barrier()

    pltpu.sync_copy(acc_sh.at[pl.ds(base_rows, ROWS_PER_TILE)],
                    agg3.at[c].at[pl.ds(base_rows, ROWS_PER_TILE)])
    pltpu.sync_copy(ws_sh.at[pl.ds(base_rows, ROWS_PER_TILE)],
                    ws_out.at[c].at[pl.ds(base_rows, ROWS_PER_TILE)])


_sc_agg = functools.partial(
    pl.kernel,
    out_type=(jax.ShapeDtypeStruct((2, NP, DH), jnp.float32),
              jax.ShapeDtypeStruct((2, NP), jnp.float32)),
    mesh=_mesh,
    scratch_types=[
        pltpu.VMEM((3, B), jnp.int32),
        pltpu.VMEM((3, B), jnp.int32),
        pltpu.VMEM((3, B), jnp.int32),
        pltpu.VMEM((B,), jnp.float32),
        pltpu.VMEM((B,), jnp.float32),
        pltpu.VMEM((B, DH), jnp.float32),
        pltpu.VMEM((B, DH), jnp.float32),
        pltpu.VMEM((ROWS_PER_TILE,), jnp.float32),
        pltpu.VMEM_SHARED((NP, DH), jnp.float32),
        pltpu.VMEM_SHARED((NP,), jnp.float32),
        pltpu.SemaphoreType.DMA,
        pltpu.SemaphoreType.DMA,
        pltpu.SemaphoreType.DMA,
        pltpu.SemaphoreType.DMA,
        pltpu.SemaphoreType.DMA,
    ],
)(_sc_agg_body)


def _dense_body(h_ref, agg_ref, ws_ref, W_ref, b_ref, out_ref):
    hl = h_ref[0]
    hh = h_ref[1]
    inv = 1.0 / (ws_ref[0] + ws_ref[1] + 1e-9)
    al = agg_ref[0] * inv
    ah = agg_ref[1] * inv
    W = W_ref[...]
    z = (jnp.dot(hl, W[0:128, :], preferred_element_type=jnp.float32)
         + jnp.dot(hh, W[128:256, :], preferred_element_type=jnp.float32)
         + jnp.dot(al, W[256:384, :], preferred_element_type=jnp.float32)
         + jnp.dot(ah, W[384:512, :], preferred_element_type=jnp.float32)
         + b_ref[...])
    z = jnp.maximum(z, 0.0)
    z = z / (jnp.sqrt(jnp.sum(z * z, axis=1, keepdims=True)) + 1e-9)
    out_ref[0, :, :] = z[:, :DH]
    out_ref[1, :, :] = z[:, DH:]


def _dense_last_body(h_ref, agg_ref, ws_ref, W_ref, b_ref, out_ref):
    hl = h_ref[0]
    hh = h_ref[1]
    inv = 1.0 / (ws_ref[0] + ws_ref[1] + 1e-9)
    al = agg_ref[0] * inv
    ah = agg_ref[1] * inv
    W = W_ref[...]
    z = (jnp.dot(hl, W[0:128, :], preferred_element_type=jnp.float32)
         + jnp.dot(hh, W[128:256, :], preferred_element_type=jnp.float32)
         + jnp.dot(al, W[256:384, :], preferred_element_type=jnp.float32)
         + jnp.dot(ah, W[384:512, :], preferred_element_type=jnp.float32)
         + b_ref[...])
    z = jnp.maximum(z, 0.0)
    z = z / (jnp.sqrt(jnp.sum(z * z, axis=1, keepdims=True)) + 1e-9)
    out_ref[...] = z


_R = 256

_dense = pl.pallas_call(
    _dense_body,
    grid=(NP // _R,),
    in_specs=[
        pl.BlockSpec((2, _R, DH), lambda i: (0, i, 0)),
        pl.BlockSpec((2, _R, DH), lambda i: (0, i, 0)),
        pl.BlockSpec((2, _R, 1), lambda i: (0, i, 0)),
        pl.BlockSpec((2 * D, D), lambda i: (0, 0)),
        pl.BlockSpec((1, D), lambda i: (0, 0)),
    ],
    out_specs=pl.BlockSpec((2, _R, DH), lambda i: (0, i, 0)),
    out_shape=jax.ShapeDtypeStruct((2, NP, DH), jnp.float32),
)


_dense_last = pl.pallas_call(
    _dense_last_body,
    grid=(NP // _R,),
    in_specs=[
        pl.BlockSpec((2, _R, DH), lambda i: (0, i, 0)),
        pl.BlockSpec((2, _R, DH), lambda i: (0, i, 0)),
        pl.BlockSpec((2, _R, 1), lambda i: (0, i, 0)),
        pl.BlockSpec((2 * D, D), lambda i: (0, 0)),
        pl.BlockSpec((1, D), lambda i: (0, 0)),
    ],
    out_specs=pl.BlockSpec((_R, D), lambda i: (i, 0)),
    out_shape=jax.ShapeDtypeStruct((NP, D), jnp.float32),
)


def kernel(x, edge_index, edge_weight, W0, b0, W1, b1):
    h3 = jnp.pad(x, ((0, NP - N), (0, 0))).reshape(NP, 2, DH).transpose(1, 0, 2)
    ei3 = jnp.concatenate(
        [edge_index,
         lax.bitcast_convert_type(edge_weight, jnp.int32)[None]], axis=0)

    agg3, ws = _sc_agg(h3, ei3)
    h3 = _dense(h3, agg3, ws.reshape(2, NP, 1), W0, b0.reshape(1, D))
    agg3, ws = _sc_agg(h3, ei3)
    out = _dense_last(h3, agg3, ws.reshape(2, NP, 1), W1, b1.reshape(1, D))
    return out[:N]
